# baseline (speedup 1.0000x reference)
import functools

import jax
import jax.numpy as jnp
from jax import lax
from jax.experimental import pallas as pl
from jax.experimental.pallas import tpu as pltpu
from jax.experimental.pallas import tpu_sc as plsc

D0, D1, D2 = 100, 100, 100
B = 16384
NC, NS, L = 2, 16, 16
NW = NC * NS
BPW = B // NW
GROUPS = BPW // L
NSTREAM = 4
CHUNK = BPW // NSTREAM

_mesh = plsc.VectorSubcoreMesh(core_axis_name="c", subcore_axis_name="s")


@functools.partial(
    pl.kernel,
    mesh=_mesh,
    out_type=jax.ShapeDtypeStruct((B,), jnp.float32),
    scratch_types=[
        pltpu.VMEM((3 * BPW,), jnp.int32),
        pltpu.VMEM((BPW,), jnp.int32),
        pltpu.VMEM((BPW,), jnp.float32),
        pltpu.SemaphoreType.DMA,
    ],
)
def _clique_gather(x_hbm, w_hbm, out_hbm, xraw_v, idx_v, val_v, sem):
    wid = lax.axis_index("s") * NC + lax.axis_index("c")
    base = wid * BPW
    pltpu.sync_copy(x_hbm.at[pl.ds(base * 3, 3 * BPW)], xraw_v)
    lanes = lax.iota(jnp.int32, L)
    e3 = lanes * 3
    perms = [(e3 + c) & (L - 1) for c in range(3)]
    srcs = [(e3 + c) >> 4 for c in range(3)]
    for g in range(GROUPS):
        v = [xraw_v[pl.ds(3 * L * g + k * L, L)] for k in range(3)]
        flat = None
        for c, wgt in ((0, D1 * D2), (1, D2), (2, 1)):
            picks = [v[k].at[perms[c]].get(mode="promise_in_bounds")
                     for k in range(3)]
            comp = jnp.where(srcs[c] == 0, picks[0],
                             jnp.where(srcs[c] == 1, picks[1], picks[2]))
            flat = comp * wgt if flat is None else flat + comp * wgt
        idx_v[pl.ds(g * L, L)] = flat
    pltpu.async_copy(w_hbm.at[idx_v], val_v, sem).wait()
    pltpu.sync_copy(val_v, out_hbm.at[pl.ds(base, BPW)])


def kernel(x, W):
    xf = x.reshape(-1).astype(jnp.int32)
    wf = W.reshape(-1)
    return _clique_gather(xf, wf).reshape(B, 1)


# speedup vs baseline: 1.2454x; 1.2454x over previous
import functools

import jax
import jax.numpy as jnp
from jax import lax
from jax.experimental import pallas as pl
from jax.experimental.pallas import tpu as pltpu
from jax.experimental.pallas import tpu_sc as plsc

D0, D1, D2 = 100, 100, 100
B = 16384
NC, NS, L = 2, 16, 16
NW = NC * NS
BPW = B // NW
GROUPS = BPW // L

_mesh = plsc.VectorSubcoreMesh(core_axis_name="c", subcore_axis_name="s")


@functools.partial(
    pl.kernel,
    mesh=_mesh,
    out_type=jax.ShapeDtypeStruct((B,), jnp.float32),
    scratch_types=[
        pltpu.VMEM((BPW,), jnp.int32),
        pltpu.VMEM((BPW,), jnp.int32),
        pltpu.VMEM((BPW,), jnp.int32),
        pltpu.VMEM((BPW,), jnp.int32),
        pltpu.VMEM((BPW,), jnp.float32),
        pltpu.SemaphoreType.DMA,
    ],
)
def _clique_gather(xt_hbm, w_hbm, out_hbm, x0_v, x1_v, x2_v, idx_v, val_v,
                   sem):
    wid = lax.axis_index("s") * NC + lax.axis_index("c")
    base = wid * BPW
    cp0 = pltpu.async_copy(xt_hbm.at[pl.ds(0 * B + base, BPW)], x0_v, sem)
    cp1 = pltpu.async_copy(xt_hbm.at[pl.ds(1 * B + base, BPW)], x1_v, sem)
    cp2 = pltpu.async_copy(xt_hbm.at[pl.ds(2 * B + base, BPW)], x2_v, sem)
    cp0.wait()
    cp1.wait()
    cp2.wait()

    def group(g, carry):
        s = pl.ds(g * L, L)
        idx_v[s] = x0_v[s] + x1_v[s] * D0 + x2_v[s] * (D0 * D1)
        return carry

    lax.fori_loop(0, GROUPS, group, 0)
    pltpu.async_copy(w_hbm.at[idx_v], val_v, sem).wait()
    pltpu.sync_copy(val_v, out_hbm.at[pl.ds(base, BPW)])


def kernel(x, W):
    xt = x.astype(jnp.int32).T.reshape(-1)
    wf = W.transpose(2, 1, 0).reshape(-1)
    return _clique_gather(xt, wf).reshape(B, 1)


# speedup vs baseline: 1.2455x; 1.0001x over previous
import functools

import jax
import jax.numpy as jnp
from jax import lax
from jax.experimental import pallas as pl
from jax.experimental.pallas import tpu as pltpu
from jax.experimental.pallas import tpu_sc as plsc

D0, D1, D2 = 100, 100, 100
B = 16384
NC, NS, L = 2, 16, 16
NW = NC * NS
BPW = B // NW
GROUPS = BPW // L

_mesh = plsc.VectorSubcoreMesh(core_axis_name="c", subcore_axis_name="s")


@functools.partial(
    pl.kernel,
    mesh=_mesh,
    out_type=jax.ShapeDtypeStruct((B,), jnp.float32),
    scratch_types=[
        pltpu.VMEM((BPW,), jnp.int32),
        pltpu.VMEM((BPW,), jnp.int32),
        pltpu.VMEM((BPW,), jnp.int32),
        pltpu.VMEM((BPW,), jnp.int32),
        pltpu.VMEM((BPW,), jnp.float32),
        pltpu.SemaphoreType.DMA,
    ],
    compiler_params=pltpu.CompilerParams(skip_device_barrier=True),
)
def _clique_gather(xt_hbm, w_hbm, out_hbm, x0_v, x1_v, x2_v, idx_v, val_v,
                   sem):
    wid = lax.axis_index("s") * NC + lax.axis_index("c")
    base = wid * BPW
    cp0 = pltpu.async_copy(xt_hbm.at[pl.ds(0 * B + base, BPW)], x0_v, sem)
    cp1 = pltpu.async_copy(xt_hbm.at[pl.ds(1 * B + base, BPW)], x1_v, sem)
    cp2 = pltpu.async_copy(xt_hbm.at[pl.ds(2 * B + base, BPW)], x2_v, sem)
    cp0.wait()
    cp1.wait()
    cp2.wait()

    def group(g, carry):
        s = pl.ds(g * L, L)
        idx_v[s] = x0_v[s] + x1_v[s] * D0 + x2_v[s] * (D0 * D1)
        return carry

    lax.fori_loop(0, GROUPS, group, 0)
    pltpu.async_copy(w_hbm.at[idx_v], val_v, sem).wait()
    pltpu.sync_copy(val_v, out_hbm.at[pl.ds(base, BPW)])


def kernel(x, W):
    xt = x.astype(jnp.int32).T.reshape(-1)
    wf = W.transpose(2, 1, 0).reshape(-1)
    return _clique_gather(xt, wf).reshape(B, 1)
